# Initial kernel scaffold; baseline (speedup 1.0000x reference)
import jax
import jax.numpy as jnp
from jax.experimental import pallas as pl


def kernel(x, edge_index, W1, b1, W2, b2):
    raise NotImplementedError("write your pallas kernel here")


# speedup vs baseline: 26.5128x; 26.5128x over previous
import functools

import jax
import jax.numpy as jnp
from jax import lax
from jax.experimental import pallas as pl
from jax.experimental.pallas import tpu as pltpu
from jax.experimental.pallas import tpu_sc as plsc

N = 10000
NPAD = 10240
E = 320000
CW = 128
NCH = 2560
EPAD = NCH * CW
NC, NS = 2, 16
NW = NC * NS
CPW = NCH // NW
RPT = NPAD // NS
BLK = 1024
D1, D2 = 128, 64
DH = 64
DW = 16

_MESH = dict(core_axis_name="c", subcore_axis_name="s", num_cores=NC,
             num_subcores=NS)


def _deg_body(dst_hbm, ones_hbm, zer_hbm, out_hbm, dst_v, ones_v, deg_sh):
    cid = lax.axis_index("c")
    sid = lax.axis_index("s")
    wid = sid * NC + cid
    pltpu.sync_copy(dst_hbm.at[pl.ds(wid * CPW, CPW)], dst_v)
    pltpu.sync_copy(ones_hbm, ones_v)

    r0 = sid * RPT
    pltpu.sync_copy(zer_hbm.at[pl.ds(r0, RPT)], deg_sh.at[pl.ds(r0, RPT)])
    plsc.subcore_barrier()

    def body(j, _):
        pltpu.sync_copy(ones_v, deg_sh.at[dst_v.at[j]], add=True)
        return 0

    lax.fori_loop(0, CPW, body, 0)
    plsc.subcore_barrier()
    pltpu.sync_copy(deg_sh.at[pl.ds(r0, RPT)], out_hbm.at[cid, pl.ds(r0, RPT)])


@functools.cache
def _deg_kernel():
    return functools.partial(
        pl.kernel,
        out_type=jax.ShapeDtypeStruct((NC, NPAD, DW), jnp.float32),
        mesh=plsc.VectorSubcoreMesh(**_MESH),
        compiler_params=pltpu.CompilerParams(use_tc_tiling_on_sc=False),
        scratch_types=[
            pltpu.VMEM((CPW, CW), jnp.int32),
            pltpu.VMEM((CW, DW), jnp.float32),
            pltpu.VMEM_SHARED((NPAD, DW), jnp.float32),
        ],
    )(_deg_body)


def _agg_body(nh, *refs):
    h_hbms = refs[:nh]
    src_hbm, dst_hbm, zer_hbm, out_hbm = refs[nh:nh + 4]
    src_v, dst_v, buf0, buf1, agg_sh, sem0, sem1 = refs[nh + 4:]
    cid = lax.axis_index("c")
    sid = lax.axis_index("s")
    wid = sid * NC + cid
    base = wid * CPW
    pltpu.sync_copy(src_hbm.at[pl.ds(base, CPW)], src_v)
    pltpu.sync_copy(dst_hbm.at[pl.ds(base, CPW)], dst_v)
    r0 = sid * RPT

    for half, h_hbm in enumerate(h_hbms):
        @pl.when(cid == 0)
        def _():
            pltpu.sync_copy(h_hbm.at[pl.ds(r0, RPT)],
                            agg_sh.at[pl.ds(r0, RPT)])

        @pl.when(cid == 1)
        def _():
            pltpu.sync_copy(zer_hbm.at[pl.ds(r0, RPT)],
                            agg_sh.at[pl.ds(r0, RPT)])

        plsc.subcore_barrier()

        pltpu.async_copy(h_hbm.at[src_v.at[0]], buf0, sem0)
        pltpu.async_copy(h_hbm.at[src_v.at[1]], buf1, sem1)

        def body(i, _):
            j0 = 2 * i
            j1 = 2 * i + 1
            pltpu.make_async_copy(h_hbm.at[src_v.at[j0]], buf0, sem0).wait()
            pltpu.sync_copy(buf0, agg_sh.at[dst_v.at[j0]], add=True)
            pltpu.async_copy(h_hbm.at[src_v.at[jnp.minimum(j0 + 2, CPW - 1)]],
                             buf0, sem0)
            pltpu.make_async_copy(h_hbm.at[src_v.at[j1]], buf1, sem1).wait()
            pltpu.sync_copy(buf1, agg_sh.at[dst_v.at[j1]], add=True)
            pltpu.async_copy(h_hbm.at[src_v.at[jnp.minimum(j1 + 2, CPW - 1)]],
                             buf1, sem1)
            return 0

        lax.fori_loop(0, CPW // 2, body, 0)
        pltpu.make_async_copy(h_hbm.at[src_v.at[CPW - 1]], buf0, sem0).wait()
        pltpu.make_async_copy(h_hbm.at[src_v.at[CPW - 1]], buf1, sem1).wait()

        plsc.subcore_barrier()
        pltpu.sync_copy(agg_sh.at[pl.ds(r0, RPT)],
                        out_hbm.at[cid, half, pl.ds(r0, RPT)])


@functools.cache
def _agg_kernel(nh):
    return functools.partial(
        pl.kernel,
        out_type=jax.ShapeDtypeStruct((NC, nh, NPAD, DH), jnp.float32),
        mesh=plsc.VectorSubcoreMesh(**_MESH),
        compiler_params=pltpu.CompilerParams(use_tc_tiling_on_sc=False),
        scratch_types=[
            pltpu.VMEM((CPW, CW), jnp.int32),
            pltpu.VMEM((CPW, CW), jnp.int32),
            pltpu.VMEM((CW, DH), jnp.float32),
            pltpu.VMEM((CW, DH), jnp.float32),
            pltpu.VMEM_SHARED((NPAD, DH), jnp.float32),
            pltpu.SemaphoreType.DMA,
            pltpu.SemaphoreType.DMA,
        ],
    )(functools.partial(_agg_body, nh))


def _tc1_body(x_ref, w_ref, d0_ref, d1_ref, ha_ref, hb_ref, dv_ref):
    d = d0_ref[:, 0:1] + d1_ref[:, 0:1] + 1.0
    dv = lax.rsqrt(d)
    h = jnp.dot(x_ref[...], w_ref[...], preferred_element_type=jnp.float32)
    h = h * dv
    ha_ref[...] = h[:, :DH]
    hb_ref[...] = h[:, DH:]
    dv_ref[...] = jnp.broadcast_to(dv, (BLK, DW))


def _tc2_body(a00_ref, a01_ref, a10_ref, a11_ref, dv_ref, b_ref, w_ref,
              o_ref):
    dv = dv_ref[:, 0:1]
    a = jnp.concatenate([a00_ref[...] + a10_ref[...],
                         a01_ref[...] + a11_ref[...]], axis=1)
    h = jnp.maximum(a * dv + b_ref[...], 0.0)
    o_ref[...] = jnp.dot(h, w_ref[...], preferred_element_type=jnp.float32) * dv


def _tc3_body(a0_ref, a1_ref, dv_ref, b_ref, o_ref):
    dv = dv_ref[:, 0:1]
    o_ref[...] = (a0_ref[...] + a1_ref[...]) * dv + b_ref[...]


def _row_spec(d):
    return pl.BlockSpec((BLK, d), lambda i: (i, 0))


def _rep_spec(r, c):
    return pl.BlockSpec((r, c), lambda i: (0, 0))


_GRID = NPAD // BLK

_tc1 = pl.pallas_call(
    _tc1_body,
    grid=(_GRID,),
    in_specs=[_row_spec(D1), _rep_spec(D1, D1), _row_spec(DW), _row_spec(DW)],
    out_specs=[_row_spec(DH), _row_spec(DH), _row_spec(DW)],
    out_shape=[jax.ShapeDtypeStruct((NPAD, DH), jnp.float32),
               jax.ShapeDtypeStruct((NPAD, DH), jnp.float32),
               jax.ShapeDtypeStruct((NPAD, DW), jnp.float32)],
)

_tc2 = pl.pallas_call(
    _tc2_body,
    grid=(_GRID,),
    in_specs=[_row_spec(DH), _row_spec(DH), _row_spec(DH), _row_spec(DH),
              _row_spec(DW), _rep_spec(1, D1), _rep_spec(D1, D2)],
    out_specs=_row_spec(D2),
    out_shape=jax.ShapeDtypeStruct((NPAD, D2), jnp.float32),
)

_tc3 = pl.pallas_call(
    _tc3_body,
    grid=(_GRID,),
    in_specs=[_row_spec(D2), _row_spec(D2), _row_spec(DW), _rep_spec(1, D2)],
    out_specs=_row_spec(D2),
    out_shape=jax.ShapeDtypeStruct((NPAD, D2), jnp.float32),
)


def kernel(x, edge_index, W1, b1, W2, b2):
    src = edge_index[0].astype(jnp.int32)
    dst = edge_index[1].astype(jnp.int32)
    pad = EPAD - E
    pad_ids = jnp.arange(pad, dtype=jnp.int32)
    src_p = jnp.concatenate([src, pad_ids % N]).reshape(NCH, CW)
    dst_p = jnp.concatenate([dst, N + pad_ids % (NPAD - N)]).reshape(NCH, CW)

    x_p = jnp.pad(x, ((0, NPAD - N), (0, 0)))
    zdeg = jnp.zeros((NPAD, DW), jnp.float32)
    ones = jnp.zeros((CW, DW), jnp.float32).at[:, 0].set(1.0)
    zer = jnp.zeros((NPAD, DH), jnp.float32)

    deg = _deg_kernel()(dst_p, ones, zdeg)
    h1a, h1b, dinv = _tc1(x_p, W1, deg[0], deg[1])
    agg1 = _agg_kernel(2)(h1a, h1b, src_p, dst_p, zer)
    h2p = _tc2(agg1[0, 0], agg1[0, 1], agg1[1, 0], agg1[1, 1], dinv,
               b1.reshape(1, D1), W2)
    agg2 = _agg_kernel(1)(h2p, src_p, dst_p, zer)
    out = _tc3(agg2[0, 0], agg2[1, 0], dinv, b2.reshape(1, D2))
    return out[:N]


# speedup vs baseline: 28.6786x; 1.0817x over previous
import functools

import jax
import jax.numpy as jnp
from jax import lax
from jax.experimental import pallas as pl
from jax.experimental.pallas import tpu as pltpu
from jax.experimental.pallas import tpu_sc as plsc

N = 10000
NPAD = 10240
E = 320000
CW = 128
NCH = 2560
EPAD = NCH * CW
NC, NS = 2, 16
NW = NC * NS
CPW = NCH // NW
RPT = NPAD // NS
BLK = 1024
D1, D2 = 128, 64
DH = 64
DW = 16

_MESH = dict(core_axis_name="c", subcore_axis_name="s", num_cores=NC,
             num_subcores=NS)


def _deg_body(dst_hbm, ones_hbm, zer_hbm, out_hbm, dst_v, ones_v, deg_sh):
    cid = lax.axis_index("c")
    sid = lax.axis_index("s")
    wid = sid * NC + cid
    pltpu.sync_copy(dst_hbm.at[pl.ds(wid * CPW, CPW)], dst_v)
    pltpu.sync_copy(ones_hbm, ones_v)

    r0 = sid * RPT
    pltpu.sync_copy(zer_hbm.at[pl.ds(r0, RPT)], deg_sh.at[pl.ds(r0, RPT)])
    plsc.subcore_barrier()

    def body(j, _):
        pltpu.sync_copy(ones_v, deg_sh.at[dst_v.at[j]], add=True)
        return 0

    lax.fori_loop(0, CPW, body, 0)
    plsc.subcore_barrier()
    pltpu.sync_copy(deg_sh.at[pl.ds(r0, RPT)], out_hbm.at[cid, pl.ds(r0, RPT)])


@functools.cache
def _deg_kernel():
    return functools.partial(
        pl.kernel,
        out_type=jax.ShapeDtypeStruct((NC, NPAD, DW), jnp.float32),
        mesh=plsc.VectorSubcoreMesh(**_MESH),
        compiler_params=pltpu.CompilerParams(use_tc_tiling_on_sc=False),
        scratch_types=[
            pltpu.VMEM((CPW, CW), jnp.int32),
            pltpu.VMEM((CW, DW), jnp.float32),
            pltpu.VMEM_SHARED((NPAD, DW), jnp.float32),
        ],
    )(_deg_body)


def _agg_body(nh, *refs):
    h_hbms = refs[:nh]
    src_hbm, dst_hbm, zer_hbm, out_hbm = refs[nh:nh + 4]
    src_v, dst_v, buf0, buf1, agg_sh, sem0, sem1 = refs[nh + 4:]
    cid = lax.axis_index("c")
    sid = lax.axis_index("s")
    wid = sid * NC + cid
    base = wid * CPW
    pltpu.sync_copy(src_hbm.at[pl.ds(base, CPW)], src_v)
    pltpu.sync_copy(dst_hbm.at[pl.ds(base, CPW)], dst_v)
    r0 = sid * RPT

    for half, h_hbm in enumerate(h_hbms):
        @pl.when(cid == 0)
        def _():
            pltpu.sync_copy(h_hbm.at[pl.ds(r0, RPT)],
                            agg_sh.at[pl.ds(r0, RPT)])

        @pl.when(cid == 1)
        def _():
            pltpu.sync_copy(zer_hbm.at[pl.ds(r0, RPT)],
                            agg_sh.at[pl.ds(r0, RPT)])

        plsc.subcore_barrier()

        pltpu.async_copy(h_hbm.at[src_v.at[0]], buf0, sem0)
        pltpu.async_copy(h_hbm.at[src_v.at[1]], buf1, sem1)

        def body(i, _):
            j0 = 2 * i
            j1 = 2 * i + 1
            pltpu.make_async_copy(h_hbm.at[src_v.at[j0]], buf0, sem0).wait()
            pltpu.sync_copy(buf0, agg_sh.at[dst_v.at[j0]], add=True)
            pltpu.async_copy(h_hbm.at[src_v.at[jnp.minimum(j0 + 2, CPW - 1)]],
                             buf0, sem0)
            pltpu.make_async_copy(h_hbm.at[src_v.at[j1]], buf1, sem1).wait()
            pltpu.sync_copy(buf1, agg_sh.at[dst_v.at[j1]], add=True)
            pltpu.async_copy(h_hbm.at[src_v.at[jnp.minimum(j1 + 2, CPW - 1)]],
                             buf1, sem1)
            return 0

        lax.fori_loop(0, CPW // 2, body, 0)
        pltpu.make_async_copy(h_hbm.at[src_v.at[CPW - 1]], buf0, sem0).wait()
        pltpu.make_async_copy(h_hbm.at[src_v.at[CPW - 1]], buf1, sem1).wait()

        plsc.subcore_barrier()
        pltpu.sync_copy(agg_sh.at[pl.ds(r0, RPT)],
                        out_hbm.at[cid, half, pl.ds(r0, RPT)])


@functools.cache
def _agg_kernel(nh):
    return functools.partial(
        pl.kernel,
        out_type=jax.ShapeDtypeStruct((NC, nh, NPAD, DH), jnp.float32),
        mesh=plsc.VectorSubcoreMesh(**_MESH),
        compiler_params=pltpu.CompilerParams(use_tc_tiling_on_sc=False),
        scratch_types=[
            pltpu.VMEM((CPW, CW), jnp.int32),
            pltpu.VMEM((CPW, CW), jnp.int32),
            pltpu.VMEM((CW, DH), jnp.float32),
            pltpu.VMEM((CW, DH), jnp.float32),
            pltpu.VMEM_SHARED((NPAD, DH), jnp.float32),
            pltpu.SemaphoreType.DMA,
            pltpu.SemaphoreType.DMA,
        ],
    )(functools.partial(_agg_body, nh))


def _tc0_body(x_ref, w_ref, u_ref):
    u_ref[...] = jnp.dot(x_ref[...], w_ref[...],
                         preferred_element_type=jnp.float32)


def _tc1_body(u_ref, deg_ref, ha_ref, hb_ref, dv_ref):
    d = deg_ref[0, :, 0:1] + deg_ref[1, :, 0:1] + 1.0
    dv = lax.rsqrt(d)
    h = u_ref[...] * dv
    ha_ref[...] = h[:, :DH]
    hb_ref[...] = h[:, DH:]
    dv_ref[...] = jnp.broadcast_to(dv, (TBLK, DW))


def _tc2_body(a_ref, dv_ref, b_ref, w_ref, o_ref):
    dv = dv_ref[:, 0:1]
    a = jnp.concatenate([a_ref[0, 0] + a_ref[1, 0],
                         a_ref[0, 1] + a_ref[1, 1]], axis=1)
    h = jnp.maximum(a * dv + b_ref[...], 0.0)
    o_ref[...] = jnp.dot(h, w_ref[...], preferred_element_type=jnp.float32) * dv


def _tc3_body(a_ref, dv_ref, b_ref, o_ref):
    dv = dv_ref[:, 0:1]
    o_ref[...] = (a_ref[0, 0] + a_ref[1, 0]) * dv + b_ref[...]


TBLK = 1000
_GRID = N // TBLK


def _row_spec(d):
    return pl.BlockSpec((TBLK, d), lambda i: (i, 0))


def _rep_spec(r, c):
    return pl.BlockSpec((r, c), lambda i: (0, 0))


def _agg_spec(nh):
    return pl.BlockSpec((NC, nh, TBLK, DH), lambda i: (0, 0, i, 0))


_tc0 = pl.pallas_call(
    _tc0_body,
    grid=(_GRID,),
    in_specs=[_row_spec(D1), _rep_spec(D1, D1)],
    out_specs=_row_spec(D1),
    out_shape=jax.ShapeDtypeStruct((N, D1), jnp.float32),
)

_tc1 = pl.pallas_call(
    _tc1_body,
    grid=(_GRID,),
    in_specs=[_row_spec(D1),
              pl.BlockSpec((NC, TBLK, DW), lambda i: (0, i, 0))],
    out_specs=[_row_spec(DH), _row_spec(DH), _row_spec(DW)],
    out_shape=[jax.ShapeDtypeStruct((NPAD, DH), jnp.float32),
               jax.ShapeDtypeStruct((NPAD, DH), jnp.float32),
               jax.ShapeDtypeStruct((N, DW), jnp.float32)],
)

_tc2 = pl.pallas_call(
    _tc2_body,
    grid=(_GRID,),
    in_specs=[_agg_spec(2), _row_spec(DW), _rep_spec(1, D1),
              _rep_spec(D1, D2)],
    out_specs=_row_spec(D2),
    out_shape=jax.ShapeDtypeStruct((NPAD, D2), jnp.float32),
)

_tc3 = pl.pallas_call(
    _tc3_body,
    grid=(_GRID,),
    in_specs=[_agg_spec(1), _row_spec(DW), _rep_spec(1, D2)],
    out_specs=_row_spec(D2),
    out_shape=jax.ShapeDtypeStruct((N, D2), jnp.float32),
)


def kernel(x, edge_index, W1, b1, W2, b2):
    src = edge_index[0].astype(jnp.int32)
    dst = edge_index[1].astype(jnp.int32)
    pad = EPAD - E
    pad_ids = jnp.arange(pad, dtype=jnp.int32)
    src_p = jnp.concatenate([src, pad_ids % N]).reshape(NCH, CW)
    dst_p = jnp.concatenate([dst, N + pad_ids % (NPAD - N)]).reshape(NCH, CW)

    zdeg = jnp.zeros((NPAD, DW), jnp.float32)
    ones = jnp.zeros((CW, DW), jnp.float32).at[:, 0].set(1.0)
    zer = jnp.zeros((NPAD, DH), jnp.float32)

    u = _tc0(x, W1)
    deg = _deg_kernel()(dst_p, ones, zdeg)
    h1a, h1b, dinv = _tc1(u, deg)
    agg1 = _agg_kernel(2)(h1a, h1b, src_p, dst_p, zer)
    h2p = _tc2(agg1, dinv, b1.reshape(1, D1), W2)
    agg2 = _agg_kernel(1)(h2p, src_p, dst_p, zer)
    return _tc3(agg2, dinv, b2.reshape(1, D2))


# speedup vs baseline: 33.4029x; 1.1647x over previous
import functools

import jax
import jax.numpy as jnp
from jax import lax
from jax.experimental import pallas as pl
from jax.experimental.pallas import tpu as pltpu
from jax.experimental.pallas import tpu_sc as plsc

N = 10000
E = 320000
CW = 128
NCH = E // CW
NC, NS = 2, 16
NW = NC * NS
CPW = NCH // NW
NEXTRA = NCH - CPW * NW
RPT = N // NS
D1, D2 = 128, 64
DW = 16
NH = N // 2
DROWS = NH + 8

_MESH = dict(core_axis_name="c", subcore_axis_name="s", num_cores=NC,
             num_subcores=NS)


def _worker_id():
    return lax.axis_index("s") * NC + lax.axis_index("c")


def _chunk_base(wid):
    return CPW * wid + jnp.maximum(wid - (NW - NEXTRA), 0)


def _deg_body(edge_hbm, onesa_hbm, onesb_hbm, zer_hbm, out_hbm,
              dst_v, idxa_v, idxb_v, onesa_v, onesb_v, deg_sh):
    cid = lax.axis_index("c")
    sid = lax.axis_index("s")
    wid = _worker_id()
    base = _chunk_base(wid)
    pltpu.sync_copy(edge_hbm.at[1, pl.ds(base, CPW + 1)], dst_v)
    pltpu.sync_copy(onesa_hbm, onesa_v)
    pltpu.sync_copy(onesb_hbm, onesb_v)

    drpt = DROWS // NS
    r0 = sid * drpt
    pltpu.sync_copy(zer_hbm.at[pl.ds(r0, drpt)], deg_sh.at[pl.ds(r0, drpt)])
    plsc.subcore_barrier()

    dump = NH + (lax.iota(jnp.int32, 16) & 7)

    def chunk(j):
        for g in range(CW // 16):
            d = dst_v[j, pl.ds(g * 16, 16)]
            lo = d < NH
            idxa_v[pl.ds(g * 16, 16)] = jnp.where(lo, d, dump)
            idxb_v[pl.ds(g * 16, 16)] = jnp.where(lo, dump, d - NH)
        pltpu.sync_copy(onesa_v, deg_sh.at[idxa_v], add=True)
        pltpu.sync_copy(onesb_v, deg_sh.at[idxb_v], add=True)

    def body(j, _):
        chunk(j)
        return 0

    lax.fori_loop(0, CPW, body, 0)

    @pl.when(wid >= NW - NEXTRA)
    def _():
        chunk(CPW)

    plsc.subcore_barrier()
    pltpu.sync_copy(deg_sh.at[pl.ds(r0, drpt)],
                    out_hbm.at[cid, pl.ds(r0, drpt)])


@functools.cache
def _deg_kernel():
    return functools.partial(
        pl.kernel,
        out_type=jax.ShapeDtypeStruct((NC, DROWS, DW), jnp.float32),
        mesh=plsc.VectorSubcoreMesh(**_MESH),
        compiler_params=pltpu.CompilerParams(use_tc_tiling_on_sc=False),
        scratch_types=[
            pltpu.VMEM((CPW + 1, CW), jnp.int32),
            pltpu.VMEM((CW,), jnp.int32),
            pltpu.VMEM((CW,), jnp.int32),
            pltpu.VMEM((CW, DW), jnp.float32),
            pltpu.VMEM((CW, DW), jnp.float32),
            pltpu.VMEM_SHARED((DROWS, DW), jnp.float32),
        ],
    )(_deg_body)


def _agg_body(D, h_hbm, edge_hbm, zer_hbm, out_hbm,
              src_v, dstb, buf0, buf1, agg_sh, sem0, sem1, semd0, semd1):
    cid = lax.axis_index("c")
    sid = lax.axis_index("s")
    wid = _worker_id()
    base = _chunk_base(wid)
    pltpu.sync_copy(edge_hbm.at[0, pl.ds(base, CPW + 1)], src_v)
    r0 = sid * RPT

    @pl.when(cid == 0)
    def _():
        pltpu.sync_copy(h_hbm.at[pl.ds(r0, RPT)], agg_sh.at[pl.ds(r0, RPT)])

    @pl.when(cid == 1)
    def _():
        pltpu.sync_copy(zer_hbm.at[pl.ds(r0, RPT)], agg_sh.at[pl.ds(r0, RPT)])

    plsc.subcore_barrier()

    pltpu.async_copy(h_hbm.at[src_v.at[0]], buf0, sem0)
    pltpu.async_copy(h_hbm.at[src_v.at[1]], buf1, sem1)
    pltpu.async_copy(edge_hbm.at[1, base + 0], dstb.at[0], semd0)
    pltpu.async_copy(edge_hbm.at[1, base + 1], dstb.at[1], semd1)

    def body(i, _):
        j0 = 2 * i
        j1 = 2 * i + 1
        pltpu.make_async_copy(h_hbm.at[src_v.at[j0]], buf0, sem0).wait()
        pltpu.make_async_copy(edge_hbm.at[1, base], dstb.at[0], semd0).wait()
        pltpu.sync_copy(buf0, agg_sh.at[dstb.at[0]], add=True)
        jn0 = jnp.minimum(j0 + 2, CPW - 1)
        pltpu.async_copy(h_hbm.at[src_v.at[jn0]], buf0, sem0)
        pltpu.async_copy(edge_hbm.at[1, base + jn0], dstb.at[0], semd0)
        pltpu.make_async_copy(h_hbm.at[src_v.at[j1]], buf1, sem1).wait()
        pltpu.make_async_copy(edge_hbm.at[1, base], dstb.at[1], semd1).wait()
        pltpu.sync_copy(buf1, agg_sh.at[dstb.at[1]], add=True)
        jn1 = jnp.minimum(j1 + 2, CPW - 1)
        pltpu.async_copy(h_hbm.at[src_v.at[jn1]], buf1, sem1)
        pltpu.async_copy(edge_hbm.at[1, base + jn1], dstb.at[1], semd1)
        return 0

    lax.fori_loop(0, CPW // 2, body, 0)
    pltpu.make_async_copy(h_hbm.at[src_v.at[CPW - 1]], buf0, sem0).wait()
    pltpu.make_async_copy(h_hbm.at[src_v.at[CPW - 1]], buf1, sem1).wait()
    pltpu.make_async_copy(edge_hbm.at[1, base], dstb.at[0], semd0).wait()
    pltpu.make_async_copy(edge_hbm.at[1, base], dstb.at[1], semd1).wait()

    @pl.when(wid >= NW - NEXTRA)
    def _():
        pltpu.sync_copy(edge_hbm.at[1, base + CPW], dstb.at[0])
        pltpu.sync_copy(h_hbm.at[src_v.at[CPW]], buf0)
        pltpu.sync_copy(buf0, agg_sh.at[dstb.at[0]], add=True)

    plsc.subcore_barrier()
    pltpu.sync_copy(agg_sh.at[pl.ds(r0, RPT)], out_hbm.at[cid, pl.ds(r0, RPT)])


@functools.cache
def _agg_kernel(D):
    return functools.partial(
        pl.kernel,
        out_type=jax.ShapeDtypeStruct((NC, N, D), jnp.float32),
        mesh=plsc.VectorSubcoreMesh(**_MESH),
        compiler_params=pltpu.CompilerParams(use_tc_tiling_on_sc=False),
        scratch_types=[
            pltpu.VMEM((CPW + 1, CW), jnp.int32),
            pltpu.VMEM((2, CW), jnp.int32),
            pltpu.VMEM((CW, D), jnp.float32),
            pltpu.VMEM((CW, D), jnp.float32),
            pltpu.VMEM_SHARED((N, D), jnp.float32),
            pltpu.SemaphoreType.DMA,
            pltpu.SemaphoreType.DMA,
            pltpu.SemaphoreType.DMA,
            pltpu.SemaphoreType.DMA,
        ],
    )(functools.partial(_agg_body, D))


def _tc0_body(x_ref, w_ref, u_ref):
    u_ref[...] = jnp.dot(x_ref[...], w_ref[...],
                         preferred_element_type=jnp.float32)


def _tc1_body(u_ref, deg_ref, h_ref, dv_ref):
    dd = deg_ref[...]
    lo = jnp.sum(dd[:, :, 0:8], axis=(0, 2))
    hi = jnp.sum(dd[:, :, 8:16], axis=(0, 2))
    blk = pl.program_id(0)
    d = jnp.where(blk < _GRID // 2, lo, hi)[:, None] + 1.0
    dv = lax.rsqrt(d)
    h_ref[...] = u_ref[...] * dv
    dv_ref[...] = jnp.broadcast_to(dv, (TBLK, 8))


def _tc2_body(a_ref, dv_ref, b_ref, w_ref, o_ref):
    dv = dv_ref[:, 0:1]
    h = jnp.maximum((a_ref[0] + a_ref[1]) * dv + b_ref[...], 0.0)
    o_ref[...] = jnp.dot(h, w_ref[...], preferred_element_type=jnp.float32) * dv


def _tc3_body(a_ref, dv_ref, b_ref, o_ref):
    dv = dv_ref[:, 0:1]
    o_ref[...] = (a_ref[0] + a_ref[1]) * dv + b_ref[...]


TBLK = 1000
_GRID = N // TBLK


def _row_spec(d):
    return pl.BlockSpec((TBLK, d), lambda i: (i, 0))


def _rep_spec(r, c):
    return pl.BlockSpec((r, c), lambda i: (0, 0))


def _agg_spec(d):
    return pl.BlockSpec((NC, TBLK, d), lambda i: (0, i, 0))


_tc0 = pl.pallas_call(
    _tc0_body,
    grid=(_GRID,),
    in_specs=[_row_spec(D1), _rep_spec(D1, D1)],
    out_specs=_row_spec(D1),
    out_shape=jax.ShapeDtypeStruct((N, D1), jnp.float32),
)

_tc1 = pl.pallas_call(
    _tc1_body,
    grid=(_GRID,),
    in_specs=[_row_spec(D1),
              pl.BlockSpec((NC, TBLK, DW), lambda i: (0, i % (_GRID // 2), 0))],
    out_specs=[_row_spec(D1), _row_spec(8)],
    out_shape=[jax.ShapeDtypeStruct((N, D1), jnp.float32),
               jax.ShapeDtypeStruct((N, 8), jnp.float32)],
)

_tc2 = pl.pallas_call(
    _tc2_body,
    grid=(_GRID,),
    in_specs=[_agg_spec(D1), _row_spec(8), _rep_spec(1, D1),
              _rep_spec(D1, D2)],
    out_specs=_row_spec(D2),
    out_shape=jax.ShapeDtypeStruct((N, D2), jnp.float32),
)

_tc3 = pl.pallas_call(
    _tc3_body,
    grid=(_GRID,),
    in_specs=[_agg_spec(D2), _row_spec(8), _rep_spec(1, D2)],
    out_specs=_row_spec(D2),
    out_shape=jax.ShapeDtypeStruct((N, D2), jnp.float32),
)


def kernel(x, edge_index, W1, b1, W2, b2):
    er = edge_index.astype(jnp.int32).reshape(2, NCH, CW)

    zdeg = jnp.zeros((DROWS, DW), jnp.float32)
    onesa = jnp.zeros((CW, DW), jnp.float32).at[:, 0].set(1.0)
    onesb = jnp.zeros((CW, DW), jnp.float32).at[:, 8].set(1.0)
    zer1 = jnp.zeros((N, D1), jnp.float32)
    zer2 = jnp.zeros((N, D2), jnp.float32)

    u = _tc0(x, W1)
    deg = _deg_kernel()(er, onesa, onesb, zdeg)
    h1p, dinv = _tc1(u, deg)
    agg1 = _agg_kernel(D1)(h1p, er, zer1)
    h2p = _tc2(agg1, dinv, b1.reshape(1, D1), W2)
    agg2 = _agg_kernel(D2)(h2p, er, zer2)
    return _tc3(agg2, dinv, b2.reshape(1, D2))
